# Initial kernel scaffold; baseline (speedup 1.0000x reference)
import jax
import jax.numpy as jnp
from jax.experimental import pallas as pl


def kernel(item_ids, edge_index, segment_ids, last_nodes, emb, W0f, al0f, ar0f, W0r, al0r, ar0r, W1f, al1f, ar1f, W1r, al1r, ar1r, fcu_w, fcu_b, fcv_w, fce_w):
    raise NotImplementedError("write your pallas kernel here")


# speedup vs baseline: 6.9845x; 6.9845x over previous
import functools

import jax
import jax.numpy as jnp
from jax import lax
from jax.experimental import pallas as pl
from jax.experimental.pallas import tpu as pltpu
from jax.experimental.pallas import tpu_sc as plsc

N = 10000
E = 160000
D = 128
H = 8
B = 512
NP = 10240
NC = 2
NS = 16
NW = NC * NS
EC = E // NS
NSL = NP // NS
HPC = H // NC
BN = 1024
NB = NP // BN
NPH = NP // 2
ACCR = NPH + 16
NTS = NPH // NS

_mesh = lambda: plsc.VectorSubcoreMesh(core_axis_name="c", subcore_axis_name="s")


def _make_gather(rows_total, d):
    b_per_w = rows_total // NW

    @functools.partial(
        pl.kernel,
        mesh=_mesh(),
        compiler_params=pltpu.CompilerParams(needs_layout_passes=False),
        out_type=jax.ShapeDtypeStruct((rows_total, d), jnp.float32),
        scratch_types=[
            pltpu.VMEM((b_per_w,), jnp.int32),
            pltpu.VMEM((b_per_w, d), jnp.float32),
            pltpu.SemaphoreType.DMA,
        ],
    )
    def k(table_hbm, idx_hbm, out_hbm, idx_v, rows_v, sem):
        wid = lax.axis_index("s") * NC + lax.axis_index("c")
        base = wid * b_per_w
        pltpu.sync_copy(idx_hbm.at[pl.ds(base, b_per_w)], idx_v)
        pltpu.async_copy(table_hbm.at[idx_v], rows_v, sem).wait()
        pltpu.sync_copy(rows_v, out_hbm.at[pl.ds(base, b_per_w)])

    return k


_gather_emb = _make_gather(NP, D)
_gather_last = _make_gather(B, D)


@functools.partial(
    pl.kernel,
    mesh=_mesh(),
    compiler_params=pltpu.CompilerParams(needs_layout_passes=False),
    out_type=jax.ShapeDtypeStruct((H * NP, D), jnp.float32),
    scratch_types=[
        pltpu.VMEM((EC,), jnp.int32),
        pltpu.VMEM((EC,), jnp.int32),
        pltpu.VMEM((EC,), jnp.float32),
        pltpu.VMEM((NP,), jnp.float32),
        pltpu.VMEM((NP,), jnp.float32),
        pltpu.VMEM((NP,), jnp.float32),
        pltpu.VMEM((NP,), jnp.float32),
        pltpu.VMEM((NSL,), jnp.float32),
        pltpu.VMEM((NSL,), jnp.float32),
        pltpu.VMEM((16, D), jnp.float32),
        pltpu.VMEM((64, D), jnp.float32),
        pltpu.VMEM_SHARED((NS * NPH,), jnp.float32),
        pltpu.VMEM_SHARED((NP,), jnp.float32),
        pltpu.VMEM_SHARED((ACCR, D), jnp.float32),
        pltpu.SemaphoreType.DMA,
    ],
)
def _edge_kernel(src_hbm, dst_hbm, el_hbm, er_hbm, h_hbm, out_hbm,
                 srcb, dstb, exb, elb, erb, dloc, dful, tbuf, sbuf,
                 rows, zb, dgrid, dfin, acc, sem):
    c = lax.axis_index("c")
    s = lax.axis_index("s")
    eb = s * EC
    nb = s * NSL
    zeros16 = jnp.zeros((16,), jnp.float32)

    pltpu.sync_copy(src_hbm.at[pl.ds(eb, EC)], srcb)
    pltpu.sync_copy(dst_hbm.at[pl.ds(eb, EC)], dstb)

    def zzb(i, _):
        for m in range(D // 16):
            zb[i, pl.ds(m * 16, 16)] = zeros16
        return 0
    lax.fori_loop(0, 64, zzb, 0)

    for k_ in range(HPC):
        hd = c * HPC + k_
        pltpu.sync_copy(el_hbm.at[pl.ds(hd * NP, NP)], elb)
        pltpu.sync_copy(er_hbm.at[pl.ds(hd * NP, NP)], erb)

        def zd(i, _):
            dloc[pl.ds(i * 16, 16)] = zeros16
            return 0
        lax.fori_loop(0, NP // 16, zd, 0)

        def p1(i, _):
            s16 = srcb[pl.ds(i * 16, 16)]
            d16 = dstb[pl.ds(i * 16, 16)]
            va = plsc.load_gather(elb, [s16])
            vb = plsc.load_gather(erb, [d16])
            e16 = va + vb
            e16 = jnp.maximum(e16, 0.2 * e16)
            ex16 = jnp.exp(e16)
            exb[pl.ds(i * 16, 16)] = ex16
            plsc.addupdate_scatter(dloc, [d16], ex16)
            return 0
        lax.fori_loop(0, EC // 16, p1, 0)

        for q2 in range(2):
            pltpu.sync_copy(dloc.at[pl.ds(q2 * NPH, NPH)],
                            dgrid.at[pl.ds(s * NPH, NPH)])
            plsc.subcore_barrier()

            def zs(j, _):
                sbuf[pl.ds(j * 16, 16)] = zeros16
                return 0
            lax.fori_loop(0, NTS // 16, zs, 0)

            def rt(t, _):
                pltpu.sync_copy(dgrid.at[pl.ds(t * NPH + s * NTS, NTS)],
                                tbuf.at[pl.ds(0, NTS)])

                def aj(j, _):
                    sbuf[pl.ds(j * 16, 16)] = (sbuf[pl.ds(j * 16, 16)]
                                               + tbuf[pl.ds(j * 16, 16)])
                    return 0
                lax.fori_loop(0, NTS // 16, aj, 0)
                return 0
            lax.fori_loop(0, NS, rt, 0)

            pltpu.sync_copy(sbuf.at[pl.ds(0, NTS)],
                            dfin.at[pl.ds(q2 * NPH + s * NTS, NTS)])
            plsc.subcore_barrier()
        pltpu.sync_copy(dfin, dful)

        for q in range(2):
            for z in range(NTS // 64):
                pltpu.sync_copy(zb, acc.at[pl.ds(s * NTS + z * 64, 64)])

            @pl.when(s == 0)
            def _():
                pltpu.sync_copy(zb.at[pl.ds(0, 16)],
                                acc.at[pl.ds(NPH, 16)])
            plsc.subcore_barrier()

            def p2(i, _):
                s16 = srcb[pl.ds(i * 16, 16)]
                d16 = dstb[pl.ds(i * 16, 16)]
                gidx = s16 + hd * NP
                pltpu.async_copy(h_hbm.at[gidx], rows, sem).wait()
                vd = plsc.load_gather(dful, [d16])
                w16 = exb[pl.ds(i * 16, 16)] / (vd + 1e-9)
                for j in range(16):
                    wj = w16[j]
                    for m in range(D // 16):
                        rows[j, pl.ds(m * 16, 16)] = (rows[j, pl.ds(m * 16, 16)]
                                                      * wj)
                dq = d16 - q * NPH
                inr = (dq >= 0) & (dq < NPH)
                idxs = jnp.where(inr, dq, NPH)
                pltpu.sync_copy(rows, acc.at[idxs], add=True)
                return 0
            lax.fori_loop(0, EC // 16, p2, 0)

            plsc.subcore_barrier()
            pltpu.sync_copy(acc.at[pl.ds(s * NTS, NTS)],
                            out_hbm.at[pl.ds(hd * NP + q * NPH + s * NTS, NTS)])
            plsc.subcore_barrier()


def _mm_heads(x, w_heads):
    g = w_heads.shape[0]

    def body(x_ref, w_ref, o_ref):
        o_ref[0] = jnp.dot(x_ref[...], w_ref[0],
                           preferred_element_type=jnp.float32)

    return pl.pallas_call(
        body,
        grid=(g, NB),
        in_specs=[
            pl.BlockSpec((BN, D), lambda j, i: (i, 0)),
            pl.BlockSpec((1, D, D), lambda j, i: (j, 0, 0)),
        ],
        out_specs=pl.BlockSpec((1, BN, D), lambda j, i: (j, i, 0)),
        out_shape=jax.ShapeDtypeStruct((g, NP, D), jnp.float32),
    )(x, w_heads)


def _mm_small(x, w):
    k = w.shape[1]

    def body(x_ref, w_ref, o_ref):
        o_ref[...] = jnp.dot(x_ref[...], w_ref[...],
                             preferred_element_type=jnp.float32)

    return pl.pallas_call(
        body,
        grid=(NB,),
        in_specs=[
            pl.BlockSpec((BN, D), lambda i: (i, 0)),
            pl.BlockSpec((D, k), lambda i: (0, 0)),
        ],
        out_specs=pl.BlockSpec((BN, k), lambda i: (i, 0)),
        out_shape=jax.ShapeDtypeStruct((NP, k), jnp.float32),
    )(x, w)


def _segsum(s_onehot, x):
    def body(s_ref, x_ref, o_ref):
        @pl.when(pl.program_id(0) == 0)
        def _():
            o_ref[...] = jnp.zeros_like(o_ref)
        o_ref[...] += lax.dot_general(
            s_ref[...], x_ref[...], (((0,), (0,)), ((), ())),
            preferred_element_type=jnp.float32)

    return pl.pallas_call(
        body,
        grid=(NB,),
        in_specs=[
            pl.BlockSpec((BN, B), lambda i: (i, 0)),
            pl.BlockSpec((BN, D), lambda i: (i, 0)),
        ],
        out_specs=pl.BlockSpec((B, D), lambda i: (0, 0)),
        out_shape=jax.ShapeDtypeStruct((B, D), jnp.float32),
    )(s_onehot, x)


def _layer_update(s_onehot, seg, cntb, agg_f, agg_r, x):
    def body(s_ref, seg_ref, cnt_ref, af_ref, ar_ref, x_ref, o_ref):
        ms = seg_ref[...] / (cnt_ref[...] + 1e-9)
        mf = af_ref[0]
        mr = ar_ref[0]
        for t in range(1, H):
            mf = jnp.maximum(mf, af_ref[t])
            mr = jnp.maximum(mr, ar_ref[t])
        o_ref[...] = (jnp.dot(s_ref[...], ms,
                              preferred_element_type=jnp.float32)
                      + mf + mr + 2.0 * x_ref[...])

    return pl.pallas_call(
        body,
        grid=(NB,),
        in_specs=[
            pl.BlockSpec((BN, B), lambda i: (i, 0)),
            pl.BlockSpec((B, D), lambda i: (0, 0)),
            pl.BlockSpec((B, D), lambda i: (0, 0)),
            pl.BlockSpec((H, BN, D), lambda i: (0, i, 0)),
            pl.BlockSpec((H, BN, D), lambda i: (0, i, 0)),
            pl.BlockSpec((BN, D), lambda i: (i, 0)),
        ],
        out_specs=pl.BlockSpec((BN, D), lambda i: (i, 0)),
        out_shape=jax.ShapeDtypeStruct((NP, D), jnp.float32),
    )(s_onehot, seg, cntb, agg_f, agg_r, x)


def _readout_escore(x, s_onehot, v512, fcu_w, fcu_b2, fce2):
    def body(x_ref, s_ref, v_ref, wu_ref, bu_ref, fe_ref, e_ref, m_ref):
        z = (jnp.dot(x_ref[...], wu_ref[...],
                     preferred_element_type=jnp.float32)
             + bu_ref[...]
             + jnp.dot(s_ref[...], v_ref[...],
                       preferred_element_type=jnp.float32))
        sg = 1.0 / (1.0 + jnp.exp(-z))
        ecol = jnp.sum(sg * fe_ref[...], axis=1, keepdims=True)
        e_ref[...] = jnp.broadcast_to(ecol, (BN, D))
        masked = s_ref[...] * jnp.broadcast_to(ecol, (BN, B)) \
            + (s_ref[...] - 1.0) * 1e30
        mpart = jnp.max(masked, axis=0)
        mrow = jnp.broadcast_to(mpart[None, :], (8, B))

        @pl.when(pl.program_id(0) == 0)
        def _():
            m_ref[...] = jnp.full((8, B), -1e30, jnp.float32)
        m_ref[...] = jnp.maximum(m_ref[...], mrow)

    return pl.pallas_call(
        body,
        grid=(NB,),
        in_specs=[
            pl.BlockSpec((BN, D), lambda i: (i, 0)),
            pl.BlockSpec((BN, B), lambda i: (i, 0)),
            pl.BlockSpec((B, D), lambda i: (0, 0)),
            pl.BlockSpec((D, D), lambda i: (0, 0)),
            pl.BlockSpec((1, D), lambda i: (0, 0)),
            pl.BlockSpec((1, D), lambda i: (0, 0)),
        ],
        out_specs=[
            pl.BlockSpec((BN, D), lambda i: (i, 0)),
            pl.BlockSpec((8, B), lambda i: (0, 0)),
        ],
        out_shape=[
            jax.ShapeDtypeStruct((NP, D), jnp.float32),
            jax.ShapeDtypeStruct((8, B), jnp.float32),
        ],
    )(x, s_onehot, v512, fcu_w, fcu_b2, fce2)


def _readout_exp(e_bc, s_onehot, m_bc):
    def body(e_ref, s_ref, m_ref, ex_ref, d_ref):
        exb = jnp.exp(e_ref[...] - jnp.dot(s_ref[...], m_ref[...],
                                           preferred_element_type=jnp.float32))
        ex_ref[...] = exb

        @pl.when(pl.program_id(0) == 0)
        def _():
            d_ref[...] = jnp.zeros_like(d_ref)
        d_ref[...] += lax.dot_general(
            s_ref[...], exb, (((0,), (0,)), ((), ())),
            preferred_element_type=jnp.float32)

    return pl.pallas_call(
        body,
        grid=(NB,),
        in_specs=[
            pl.BlockSpec((BN, D), lambda i: (i, 0)),
            pl.BlockSpec((BN, B), lambda i: (i, 0)),
            pl.BlockSpec((B, D), lambda i: (0, 0)),
        ],
        out_specs=[
            pl.BlockSpec((BN, D), lambda i: (i, 0)),
            pl.BlockSpec((B, D), lambda i: (0, 0)),
        ],
        out_shape=[
            jax.ShapeDtypeStruct((NP, D), jnp.float32),
            jax.ShapeDtypeStruct((B, D), jnp.float32),
        ],
    )(e_bc, s_onehot, m_bc)


def _readout_sum(x, ex_bc, s_onehot, denom):
    def body(x_ref, ex_ref, s_ref, d_ref, o_ref):
        alpha = ex_ref[...] / (jnp.dot(s_ref[...], d_ref[...],
                                       preferred_element_type=jnp.float32)
                               + 1e-9)

        @pl.when(pl.program_id(0) == 0)
        def _():
            o_ref[...] = jnp.zeros_like(o_ref)
        o_ref[...] += lax.dot_general(
            s_ref[...], x_ref[...] * alpha, (((0,), (0,)), ((), ())),
            preferred_element_type=jnp.float32)

    return pl.pallas_call(
        body,
        grid=(NB,),
        in_specs=[
            pl.BlockSpec((BN, D), lambda i: (i, 0)),
            pl.BlockSpec((BN, D), lambda i: (i, 0)),
            pl.BlockSpec((BN, B), lambda i: (i, 0)),
            pl.BlockSpec((B, D), lambda i: (0, 0)),
        ],
        out_specs=pl.BlockSpec((B, D), lambda i: (0, 0)),
        out_shape=jax.ShapeDtypeStruct((B, D), jnp.float32),
    )(x, ex_bc, s_onehot, denom)


def kernel(item_ids, edge_index, segment_ids, last_nodes, emb,
           W0f, al0f, ar0f, W0r, al0r, ar0r,
           W1f, al1f, ar1f, W1r, al1r, ar1r,
           fcu_w, fcu_b, fcv_w, fce_w):
    src = edge_index[0].astype(jnp.int32)
    dst = edge_index[1].astype(jnp.int32)
    ids_pad = jnp.pad(item_ids.astype(jnp.int32), (0, NP - N))
    seg_pad = jnp.pad(segment_ids.astype(jnp.int32), (0, NP - N),
                      constant_values=B)
    s_onehot = (seg_pad[:, None] == jnp.arange(B, dtype=jnp.int32)[None, :]
                ).astype(jnp.float32)

    x = _gather_emb(emb, ids_pad)
    ones = jnp.ones((NP, D), jnp.float32) * (seg_pad[:, None] < B)
    cntb = _segsum(s_onehot, ones)

    layers = [(W0f, al0f, ar0f, W0r, al0r, ar0r),
              (W1f, al1f, ar1f, W1r, al1r, ar1r)]
    for (Wf, alf, arf, Wr, alr, arr) in layers:
        w_heads = jnp.concatenate([Wf.transpose(1, 0, 2),
                                   Wr.transpose(1, 0, 2)], axis=0)
        wab = jnp.concatenate([
            jnp.einsum('dhk,hk->dh', Wf, alf),
            jnp.einsum('dhk,hk->dh', Wf, arf),
            jnp.einsum('dhk,hk->dh', Wr, alr),
            jnp.einsum('dhk,hk->dh', Wr, arr)], axis=1)

        h_all = _mm_heads(x, w_heads)
        eler = _mm_small(x, wab)
        elf_t = eler[:, 0:8].T
        erf_t = eler[:, 8:16].T
        elr_t = eler[:, 16:24].T
        err_t = eler[:, 24:32].T
        h_f = h_all[:H].reshape(H * NP, D)
        h_r = h_all[H:].reshape(H * NP, D)

        agg_f = _edge_kernel(src, dst, elf_t.reshape(-1), erf_t.reshape(-1),
                             h_f).reshape(H, NP, D)
        agg_r = _edge_kernel(dst, src, elr_t.reshape(-1), err_t.reshape(-1),
                             h_r).reshape(H, NP, D)

        seg = _segsum(s_onehot, x)
        x = _layer_update(s_onehot, seg, cntb, agg_f, agg_r, x)

    xl = _gather_last(x, last_nodes.astype(jnp.int32))
    v512 = _mm_small(jnp.pad(xl, ((0, NP - B), (0, 0))), fcv_w)[:B]
    e_bc, m8 = _readout_escore(x, s_onehot, v512, fcu_w,
                               fcu_b.reshape(1, D),
                               fce_w.reshape(1, D))
    m_bc = jnp.broadcast_to(m8[0][:, None], (B, D))
    ex_bc, denom = _readout_exp(e_bc, s_onehot, m_bc)
    rst = _readout_sum(x, ex_bc, s_onehot, denom)
    return rst
